# baseline (speedup 1.0000x reference)
import functools

import jax
import jax.numpy as jnp
from jax import lax
from jax.experimental import pallas as pl

ROWS_PER_BLOCK = 8


def _xor_partner(x, d, idx):
    left = jnp.roll(x, -d, axis=1)
    right = jnp.roll(x, d, axis=1)
    return jnp.where((idx & d) == 0, left, right)


def _subloss_block_kernel(pre_ref, lab_ref, out_ref, *, log_n):
    pre = pre_ref[...]
    lab = lab_ref[...]
    r, n = pre.shape

    bits = lax.bitcast_convert_type(lab, jnp.int32)
    key = jnp.where(bits < 0, jnp.invert(bits) ^ jnp.int32(-2147483648), bits)

    idx = lax.broadcasted_iota(jnp.int32, (r, n), 1)

    def phase_body(k, carry):
        key, pre = carry
        asc_bit = (idx >> k) & 1
        kvec = jnp.broadcast_to(k, (r, n)).astype(jnp.int32)
        for j in range(log_n - 1, -1, -1):
            d = 1 << j
            key_p = _xor_partner(key, d, idx)
            pre_p = _xor_partner(pre, d, idx)
            pos_bit = (idx >> j) & 1
            want_min = asc_bit == pos_bit
            lt = key_p < key
            gt = key_p > key
            take = (want_min & lt) | (jnp.logical_not(want_min) & gt)
            active = jnp.int32(j) < kvec
            take = take & active
            key = jnp.where(take, key_p, key)
            pre = jnp.where(take, pre_p, pre)
        return key, pre

    key, pre = lax.fori_loop(1, log_n + 1, phase_body, (key, pre))

    diff = pre[:, :-1] - pre[:, 1:]
    rowsums = jnp.sum(jnp.maximum(diff, 0.0), axis=1)
    out_ref[...] = jnp.broadcast_to(rowsums[:, None], (r, 128))


@jax.jit
def kernel(uncertainty_pre, uncertainty_label):
    b, n = uncertainty_pre.shape
    log_n = n.bit_length() - 1
    assert n == (1 << log_n)
    r = min(ROWS_PER_BLOCK, b)

    out = pl.pallas_call(
        functools.partial(_subloss_block_kernel, log_n=log_n),
        grid=(b // r,),
        in_specs=[
            pl.BlockSpec((r, n), lambda i: (i, 0)),
            pl.BlockSpec((r, n), lambda i: (i, 0)),
        ],
        out_specs=pl.BlockSpec((r, 128), lambda i: (i, 0)),
        out_shape=jax.ShapeDtypeStruct((b, 128), jnp.float32),
    )(uncertainty_pre, uncertainty_label)

    return jnp.sum(out[:, 0]) / b


# speedup vs baseline: 3.5368x; 3.5368x over previous
import functools

import jax
import jax.numpy as jnp
from jax import lax
from jax.experimental import pallas as pl
from jax.experimental.pallas import tpu as pltpu
from jax.experimental.pallas import tpu_sc as plsc

N = 32768
W = 2048
NWIN = N // W
GRP = 4
NGRP = NWIN // GRP
NC = 2
NS = 16
NWORK = NC * NS
L = 16

PASSES = ((0, 2048), (11, 2048), (22, 1024))


def _to_key(l16):
    bits = plsc.bitcast(l16, jnp.int32)
    return jnp.where(bits < 0, bits ^ jnp.int32(-1),
                     bits | jnp.int32(-2147483648))


def _digit(k16, shift, nbins):
    if shift == 0:
        return k16 & jnp.int32(nbins - 1)
    d = lax.shift_right_logical(k16, jnp.int32(shift))
    if shift + (nbins.bit_length() - 1) < 32:
        d = d & jnp.int32(nbins - 1)
    return d


def _row_pass(row, pass_idx, pre_hbm, lab_hbm, key_ping, pre_ping, key_out,
              pre_out, stage_k, stage_p, cursor, hist, running):
    shift, nbins = PASSES[pass_idx]
    nbv = nbins // L
    zeros16 = jnp.zeros((L,), jnp.int32)

    def zero_body(i, _):
        for h in range(8):
            hist[pl.ds(h * 2048 + i * L, L)] = zeros16
        return 0

    lax.fori_loop(0, nbv, zero_body, 0, unroll=2)

    def hist_group(g, _):
        for wl in range(GRP):
            w = g * GRP + wl
            if pass_idx == 0:
                pltpu.sync_copy(lab_hbm.at[row, pl.ds(w * W, W)],
                                stage_p.at[wl])
            else:
                pltpu.sync_copy(key_ping.at[row, pl.ds(w * W, W)],
                                stage_k.at[wl])

        def hist_body(i, _):
            for wl in range(GRP):
                w = g * GRP + wl
                shamt = (w // 8) * 16
                incr16 = zeros16 + lax.shift_left(jnp.int32(1), shamt)
                off = (w % 8) * 2048
                if pass_idx == 0:
                    l16 = stage_p[wl, pl.ds(i * L, L)]
                    k16 = _to_key(l16)
                    stage_k[wl, pl.ds(i * L, L)] = k16
                else:
                    k16 = stage_k[wl, pl.ds(i * L, L)]
                d = _digit(k16, shift, nbins)
                plsc.addupdate_scatter(hist, [d + off], incr16)
            return 0

        lax.fori_loop(0, W // L, hist_body, 0)

        if pass_idx == 0:
            for wl in range(GRP):
                w = g * GRP + wl
                pltpu.sync_copy(stage_k.at[wl],
                                key_ping.at[row, pl.ds(w * W, W)])
        return 0

    lax.fori_loop(0, NGRP, hist_group, 0)

    def scan_body(i, carry):
        tot_pairs = hist[pl.ds(i * L, L)]
        for h in range(1, 8):
            tot_pairs = tot_pairs + hist[pl.ds(h * 2048 + i * L, L)]
        tot16 = (tot_pairs & jnp.int32(0xFFFF)) + lax.shift_right_logical(
            tot_pairs, jnp.int32(16))
        incl = plsc.cumsum(tot16)
        running[pl.ds(i * L, L)] = incl - tot16 + carry
        return carry + jnp.sum(tot16)

    lax.fori_loop(0, nbv, scan_body, jnp.int32(0))

    def perm_group(g, _):
        for wl in range(GRP):
            w = g * GRP + wl
            pltpu.sync_copy(key_ping.at[row, pl.ds(w * W, W)], stage_k.at[wl])
            if pass_idx == 0:
                pltpu.sync_copy(pre_hbm.at[row, pl.ds(w * W, W)],
                                stage_p.at[wl])
            else:
                pltpu.sync_copy(pre_ping.at[row, pl.ds(w * W, W)],
                                stage_p.at[wl])

        def build_body(i, _):
            run16 = running[pl.ds(i * L, L)]
            for wl in range(GRP):
                w = g * GRP + wl
                shamt = (w // 8) * 16
                cnt_pair = hist[pl.ds((w % 8) * 2048 + i * L, L)]
                cnt16 = lax.shift_right_logical(cnt_pair, shamt) \
                    & jnp.int32(0xFFFF)
                cursor[pl.ds(wl * nbins + i * L, L)] = run16
                run16 = run16 + cnt16
            running[pl.ds(i * L, L)] = run16
            return 0

        lax.fori_loop(0, nbv, build_body, 0)

        def perm_body(i, _):
            for wl in range(GRP):
                k16 = stage_k[wl, pl.ds(i * L, L)]
                d = _digit(k16, shift, nbins)
                cnt, last = plsc.scan_count(d)
                cidx = d + jnp.int32(wl * nbins)
                cur = plsc.load_gather(cursor, [cidx])
                pos = cur + (cnt - jnp.int32(1))
                if pass_idx != 2:
                    plsc.store_scatter(key_out, [pos], k16)
                p16 = stage_p[wl, pl.ds(i * L, L)]
                plsc.store_scatter(pre_out, [pos], p16)
                plsc.store_scatter(cursor, [cidx], cur + cnt, mask=last)
            return 0

        lax.fori_loop(0, W // L, perm_body, 0)
        return 0

    lax.fori_loop(0, NGRP, perm_group, 0)

    if pass_idx != 2:
        def back_body(w, _):
            pltpu.sync_copy(key_out.at[pl.ds(w * W, W)],
                            key_ping.at[row, pl.ds(w * W, W)])
            pltpu.sync_copy(pre_out.at[pl.ds(w * W, W)],
                            pre_ping.at[row, pl.ds(w * W, W)])
            return 0

        lax.fori_loop(0, NWIN, back_body, 0)


def _make_sc_kernel(batch):
    rows_per_worker = batch // NWORK
    mesh = plsc.VectorSubcoreMesh(core_axis_name="c", subcore_axis_name="s")

    @functools.partial(
        pl.kernel,
        mesh=mesh,
        out_type=(
            jax.ShapeDtypeStruct((NWORK, L), jnp.float32),
            jax.ShapeDtypeStruct((batch, N), jnp.int32),
            jax.ShapeDtypeStruct((batch, N), jnp.float32),
        ),
        scratch_types=[
            pltpu.VMEM((N,), jnp.int32),
            pltpu.VMEM((N,), jnp.float32),
            pltpu.VMEM((GRP, W), jnp.int32),
            pltpu.VMEM((GRP, W), jnp.float32),
            pltpu.VMEM((GRP * 2048,), jnp.int32),
            pltpu.VMEM((8 * 2048,), jnp.int32),
            pltpu.VMEM((2048,), jnp.int32),
        ],
        compiler_params=pltpu.CompilerParams(needs_layout_passes=False),
    )
    def sc_kernel(pre_hbm, lab_hbm, out_hbm, key_ping, pre_ping, key_out,
                  pre_out, stage_k, stage_p, cursor, hist, running):
        cid = lax.axis_index("c")
        sid = lax.axis_index("s")
        wid = sid * NC + cid

        iota = lax.iota(jnp.int32, L)

        def row_body(t, acc):
            row = wid * rows_per_worker + t
            for pass_idx in range(3):
                _row_pass(row, pass_idx, pre_hbm, lab_hbm, key_ping,
                          pre_ping, key_out, pre_out, stage_k, stage_p,
                          cursor, hist, running)

            def epi_body(i, acc):
                for u in range(4):
                    base = (i * 4 + u) * L
                    a16 = pre_out[pl.ds(base, L)]
                    nxt = jnp.minimum(iota + (base + 1), jnp.int32(N - 1))
                    b16 = plsc.load_gather(pre_out, [nxt])
                    acc = acc + jnp.maximum(a16 - b16, 0.0)
                return acc

            return lax.fori_loop(0, N // L // 4, epi_body, acc)

        acc = lax.fori_loop(0, rows_per_worker, row_body,
                            jnp.zeros((L,), jnp.float32))

        stage_p[0, pl.ds(0, L)] = acc
        pltpu.sync_copy(stage_p.at[0, pl.ds(0, L)], out_hbm.at[wid])

    return sc_kernel


@jax.jit
def kernel(uncertainty_pre, uncertainty_label):
    b, n = uncertainty_pre.shape
    assert n == N and b % NWORK == 0
    out, _, _ = _make_sc_kernel(b)(uncertainty_pre, uncertainty_label)
    return jnp.sum(out) / b


# speedup vs baseline: 4.9715x; 1.4056x over previous
import functools

import jax
import jax.numpy as jnp
from jax import lax
from jax.experimental import pallas as pl
from jax.experimental.pallas import tpu as pltpu
from jax.experimental.pallas import tpu_sc as plsc

N = 32768
W = 4096
NWIN = N // W
GRP = 4
NGRP = NWIN // GRP
HSLOT = NWIN // 2
NC = 2
NS = 16
NWORK = NC * NS
L = 16

PASSES = ((0, 2048), (11, 2048), (22, 1024))


def _to_key(l16):
    bits = plsc.bitcast(l16, jnp.int32)
    return jnp.where(bits < 0, bits ^ jnp.int32(-1),
                     bits | jnp.int32(-2147483648))


def _digit(k16, shift, nbins):
    if shift == 0:
        return k16 & jnp.int32(nbins - 1)
    d = lax.shift_right_logical(k16, jnp.int32(shift))
    if shift + (nbins.bit_length() - 1) < 32:
        d = d & jnp.int32(nbins - 1)
    return d


def _drain(cps):
    for cp in cps:
        cp.wait()


def _row_pass(row, pass_idx, pre_hbm, lab_hbm, key_ping, pre_ping, key_out,
              pre_out, stage_k, stage_p, cursor, hist, running, sem):
    shift, nbins = PASSES[pass_idx]
    nbv = nbins // L
    zeros16 = jnp.zeros((L,), jnp.int32)

    def zero_body(i, _):
        for h in range(HSLOT):
            hist[pl.ds(h * 2048 + i * L, L)] = zeros16
        return 0

    lax.fori_loop(0, nbv, zero_body, 0, unroll=2)

    def hist_group(g, _):
        cps = []
        for wl in range(GRP):
            w = g * GRP + wl
            if pass_idx == 0:
                cps.append(pltpu.async_copy(
                    lab_hbm.at[row, pl.ds(w * W, W)], stage_p.at[wl], sem))
            else:
                cps.append(pltpu.async_copy(
                    key_ping.at[row, pl.ds(w * W, W)], stage_k.at[wl], sem))
        _drain(cps)

        def hist_body(i, _):
            for wl in range(GRP):
                w = g * GRP + wl
                shamt = (w // HSLOT) * 16
                incr16 = zeros16 + lax.shift_left(jnp.int32(1), shamt)
                off = (w % HSLOT) * 2048
                if pass_idx == 0:
                    l16 = stage_p[wl, pl.ds(i * L, L)]
                    k16 = _to_key(l16)
                    stage_k[wl, pl.ds(i * L, L)] = k16
                else:
                    k16 = stage_k[wl, pl.ds(i * L, L)]
                d = _digit(k16, shift, nbins)
                plsc.addupdate_scatter(hist, [d + off], incr16)
            return 0

        lax.fori_loop(0, W // L, hist_body, 0)

        if pass_idx == 0:
            cps = []
            for wl in range(GRP):
                w = g * GRP + wl
                cps.append(pltpu.async_copy(
                    stage_k.at[wl], key_ping.at[row, pl.ds(w * W, W)], sem))
            _drain(cps)
        return 0

    lax.fori_loop(0, NGRP, hist_group, 0)

    def scan_body(i, carry):
        tot_pairs = hist[pl.ds(i * L, L)]
        for h in range(1, HSLOT):
            tot_pairs = tot_pairs + hist[pl.ds(h * 2048 + i * L, L)]
        tot16 = (tot_pairs & jnp.int32(0xFFFF)) + lax.shift_right_logical(
            tot_pairs, jnp.int32(16))
        incl = plsc.cumsum(tot16)
        running[pl.ds(i * L, L)] = incl - tot16 + carry
        return carry + jnp.sum(tot16)

    lax.fori_loop(0, nbv, scan_body, jnp.int32(0))

    def perm_group(g, _):
        cps = []
        for wl in range(GRP):
            w = g * GRP + wl
            cps.append(pltpu.async_copy(
                key_ping.at[row, pl.ds(w * W, W)], stage_k.at[wl], sem))
            if pass_idx == 0:
                cps.append(pltpu.async_copy(
                    pre_hbm.at[row, pl.ds(w * W, W)], stage_p.at[wl], sem))
            else:
                cps.append(pltpu.async_copy(
                    pre_ping.at[row, pl.ds(w * W, W)], stage_p.at[wl], sem))
        _drain(cps)

        def build_body(i, _):
            run16 = running[pl.ds(i * L, L)]
            for wl in range(GRP):
                w = g * GRP + wl
                shamt = (w // HSLOT) * 16
                cnt_pair = hist[pl.ds((w % HSLOT) * 2048 + i * L, L)]
                cnt16 = lax.shift_right_logical(cnt_pair, shamt) \
                    & jnp.int32(0xFFFF)
                cursor[pl.ds(wl * nbins + i * L, L)] = run16
                run16 = run16 + cnt16
            running[pl.ds(i * L, L)] = run16
            return 0

        lax.fori_loop(0, nbv, build_body, 0)

        def perm_body(i, _):
            for wl in range(GRP):
                k16 = stage_k[wl, pl.ds(i * L, L)]
                d = _digit(k16, shift, nbins)
                cnt, last = plsc.scan_count(d)
                cidx = d + jnp.int32(wl * nbins)
                cur = plsc.load_gather(cursor, [cidx])
                pos = cur + (cnt - jnp.int32(1))
                if pass_idx != 2:
                    plsc.store_scatter(key_out, [pos], k16)
                p16 = stage_p[wl, pl.ds(i * L, L)]
                plsc.store_scatter(pre_out, [pos], p16)
                plsc.store_scatter(cursor, [cidx], cur + cnt, mask=last)
            return 0

        lax.fori_loop(0, W // L, perm_body, 0)
        return 0

    lax.fori_loop(0, NGRP, perm_group, 0)

    if pass_idx != 2:
        cps = [pltpu.async_copy(key_out, key_ping.at[row], sem),
               pltpu.async_copy(pre_out, pre_ping.at[row], sem)]
        _drain(cps)


def _make_sc_kernel(batch):
    rows_per_worker = batch // NWORK
    mesh = plsc.VectorSubcoreMesh(core_axis_name="c", subcore_axis_name="s")

    @functools.partial(
        pl.kernel,
        mesh=mesh,
        out_type=(
            jax.ShapeDtypeStruct((NWORK, L), jnp.float32),
            jax.ShapeDtypeStruct((batch, N), jnp.int32),
            jax.ShapeDtypeStruct((batch, N), jnp.float32),
        ),
        scratch_types=[
            pltpu.VMEM((N,), jnp.int32),
            pltpu.VMEM((N,), jnp.float32),
            pltpu.VMEM((GRP, W), jnp.int32),
            pltpu.VMEM((GRP, W), jnp.float32),
            pltpu.VMEM((GRP * 2048,), jnp.int32),
            pltpu.VMEM((HSLOT * 2048,), jnp.int32),
            pltpu.VMEM((2048,), jnp.int32),
            pltpu.SemaphoreType.DMA,
        ],
        compiler_params=pltpu.CompilerParams(needs_layout_passes=False),
    )
    def sc_kernel(pre_hbm, lab_hbm, out_hbm, key_ping, pre_ping, key_out,
                  pre_out, stage_k, stage_p, cursor, hist, running, sem):
        cid = lax.axis_index("c")
        sid = lax.axis_index("s")
        wid = sid * NC + cid

        iota = lax.iota(jnp.int32, L)

        def row_body(t, acc):
            row = wid * rows_per_worker + t
            for pass_idx in range(3):
                _row_pass(row, pass_idx, pre_hbm, lab_hbm, key_ping,
                          pre_ping, key_out, pre_out, stage_k, stage_p,
                          cursor, hist, running, sem)

            def epi_body(i, acc):
                for u in range(4):
                    base = (i * 4 + u) * L
                    a16 = pre_out[pl.ds(base, L)]
                    nxt = jnp.minimum(iota + (base + 1), jnp.int32(N - 1))
                    b16 = plsc.load_gather(pre_out, [nxt])
                    acc = acc + jnp.maximum(a16 - b16, 0.0)
                return acc

            return lax.fori_loop(0, N // L // 4, epi_body, acc)

        acc = lax.fori_loop(0, rows_per_worker, row_body,
                            jnp.zeros((L,), jnp.float32))

        stage_p[0, pl.ds(0, L)] = acc
        pltpu.sync_copy(stage_p.at[0, pl.ds(0, L)], out_hbm.at[wid])

    return sc_kernel


@jax.jit
def kernel(uncertainty_pre, uncertainty_label):
    b, n = uncertainty_pre.shape
    assert n == N and b % NWORK == 0
    out, _, _ = _make_sc_kernel(b)(uncertainty_pre, uncertainty_label)
    return jnp.sum(out) / b


# speedup vs baseline: 5.8430x; 1.1753x over previous
import functools

import jax
import jax.numpy as jnp
from jax import lax
from jax.experimental import pallas as pl
from jax.experimental.pallas import tpu as pltpu
from jax.experimental.pallas import tpu_sc as plsc

N = 32768
W = 4096
NWIN = N // W
GRP = 4
NGRP = NWIN // GRP
HSLOT = NWIN // 2
NC = 2
NS = 16
NWORK = NC * NS
L = 16

PASSES = ((0, 2048), (11, 2048), (22, 1024))


def _to_key(l16):
    bits = plsc.bitcast(l16, jnp.int32)
    return jnp.where(bits < 0, bits ^ jnp.int32(-1),
                     bits | jnp.int32(-2147483648))


def _digit(k16, shift, nbins):
    if shift == 0:
        return k16 & jnp.int32(nbins - 1)
    d = lax.shift_right_logical(k16, jnp.int32(shift))
    if shift + (nbins.bit_length() - 1) < 32:
        d = d & jnp.int32(nbins - 1)
    return d


def _drain(cps):
    for cp in cps:
        cp.wait()


def _row_pass(row, pass_idx, pre_hbm, lab_hbm, key_ping, pre_ping, key_out,
              pre_out, stage_k, stage_p, cursor, hist, hist_next, running,
              sem):
    shift, nbins = PASSES[pass_idx]
    nbv = nbins // L
    zeros16 = jnp.zeros((L,), jnp.int32)

    def zero_body(i, _):
        for h in range(HSLOT):
            if pass_idx == 0:
                hist[pl.ds(h * 2048 + i * L, L)] = zeros16
            if hist_next is not None:
                hist_next[pl.ds(h * 2048 + i * L, L)] = zeros16
        return 0

    lax.fori_loop(0, 2048 // L, zero_body, 0, unroll=2)

    def hist_group(g, _):
        cps = []
        for wl in range(GRP):
            w = g * GRP + wl
            cps.append(pltpu.async_copy(
                lab_hbm.at[row, pl.ds(w * W, W)], stage_p.at[wl], sem))
        _drain(cps)

        def hist_body(i, _):
            for wl in range(GRP):
                w = g * GRP + wl
                shamt = (w // HSLOT) * 16
                incr16 = zeros16 + lax.shift_left(jnp.int32(1), shamt)
                off = (w % HSLOT) * 2048
                l16 = stage_p[wl, pl.ds(i * L, L)]
                k16 = _to_key(l16)
                stage_k[wl, pl.ds(i * L, L)] = k16
                d = _digit(k16, shift, nbins)
                plsc.addupdate_scatter(hist, [d + off], incr16)
            return 0

        lax.fori_loop(0, W // L, hist_body, 0)

        cps = []
        for wl in range(GRP):
            w = g * GRP + wl
            cps.append(pltpu.async_copy(
                stage_k.at[wl], key_ping.at[row, pl.ds(w * W, W)], sem))
        _drain(cps)
        return 0

    if pass_idx == 0:
        lax.fori_loop(0, NGRP, hist_group, 0)

    def scan_body(i, carry):
        tot_pairs = hist[pl.ds(i * L, L)]
        for h in range(1, HSLOT):
            tot_pairs = tot_pairs + hist[pl.ds(h * 2048 + i * L, L)]
        tot16 = (tot_pairs & jnp.int32(0xFFFF)) + lax.shift_right_logical(
            tot_pairs, jnp.int32(16))
        incl = plsc.cumsum(tot16)
        running[pl.ds(i * L, L)] = incl - tot16 + carry
        return carry + jnp.sum(tot16)

    lax.fori_loop(0, nbv, scan_body, jnp.int32(0))

    def perm_group(g, _):
        cps = []
        for wl in range(GRP):
            w = g * GRP + wl
            cps.append(pltpu.async_copy(
                key_ping.at[row, pl.ds(w * W, W)], stage_k.at[wl], sem))
            if pass_idx == 0:
                cps.append(pltpu.async_copy(
                    pre_hbm.at[row, pl.ds(w * W, W)], stage_p.at[wl], sem))
            else:
                cps.append(pltpu.async_copy(
                    pre_ping.at[row, pl.ds(w * W, W)], stage_p.at[wl], sem))
        _drain(cps)

        def build_body(i, _):
            run16 = running[pl.ds(i * L, L)]
            for wl in range(GRP):
                w = g * GRP + wl
                shamt = (w // HSLOT) * 16
                cnt_pair = hist[pl.ds((w % HSLOT) * 2048 + i * L, L)]
                cnt16 = lax.shift_right_logical(cnt_pair, shamt) \
                    & jnp.int32(0xFFFF)
                cursor[pl.ds(wl * nbins + i * L, L)] = run16
                run16 = run16 + cnt16
            running[pl.ds(i * L, L)] = run16
            return 0

        lax.fori_loop(0, nbv, build_body, 0)

        ones_lo = zeros16 + jnp.int32(1)
        ones_hi = zeros16 + jnp.int32(1 << 16)
        if hist_next is not None:
            nshift, nnbins = PASSES[pass_idx + 1]

        def perm_body(i, _):
            for wl in range(GRP):
                k16 = stage_k[wl, pl.ds(i * L, L)]
                d = _digit(k16, shift, nbins)
                cnt, last = plsc.scan_count(d)
                cidx = d + jnp.int32(wl * nbins)
                cur = plsc.load_gather(cursor, [cidx])
                pos = cur + (cnt - jnp.int32(1))
                if pass_idx != 2:
                    plsc.store_scatter(key_out, [pos], k16)
                p16 = stage_p[wl, pl.ds(i * L, L)]
                plsc.store_scatter(pre_out, [pos], p16)
                plsc.store_scatter(cursor, [cidx], cur + cnt, mask=last)
                if hist_next is not None:
                    dnext = _digit(k16, nshift, nnbins)
                    wdest = lax.shift_right_logical(pos, jnp.int32(12))
                    nidx = lax.shift_left(wdest & jnp.int32(HSLOT - 1),
                                          jnp.int32(11)) + dnext
                    lo = wdest < jnp.int32(HSLOT)
                    plsc.addupdate_scatter(hist_next, [nidx], ones_lo,
                                           mask=lo)
                    plsc.addupdate_scatter(hist_next, [nidx], ones_hi,
                                           mask=jnp.logical_not(lo))
            return 0

        lax.fori_loop(0, W // L, perm_body, 0)
        return 0

    lax.fori_loop(0, NGRP, perm_group, 0)

    if pass_idx != 2:
        cps = [pltpu.async_copy(key_out, key_ping.at[row], sem),
               pltpu.async_copy(pre_out, pre_ping.at[row], sem)]
        _drain(cps)


def _make_sc_kernel(batch):
    rows_per_worker = batch // NWORK
    mesh = plsc.VectorSubcoreMesh(core_axis_name="c", subcore_axis_name="s")

    @functools.partial(
        pl.kernel,
        mesh=mesh,
        out_type=(
            jax.ShapeDtypeStruct((NWORK, L), jnp.float32),
            jax.ShapeDtypeStruct((batch, N), jnp.int32),
            jax.ShapeDtypeStruct((batch, N), jnp.float32),
        ),
        scratch_types=[
            pltpu.VMEM((N,), jnp.int32),
            pltpu.VMEM((N,), jnp.float32),
            pltpu.VMEM((GRP, W), jnp.int32),
            pltpu.VMEM((GRP, W), jnp.float32),
            pltpu.VMEM((GRP * 2048,), jnp.int32),
            pltpu.VMEM((HSLOT * 2048,), jnp.int32),
            pltpu.VMEM((HSLOT * 2048,), jnp.int32),
            pltpu.VMEM((2048,), jnp.int32),
            pltpu.SemaphoreType.DMA,
        ],
        compiler_params=pltpu.CompilerParams(needs_layout_passes=False),
    )
    def sc_kernel(pre_hbm, lab_hbm, out_hbm, key_ping, pre_ping, key_out,
                  pre_out, stage_k, stage_p, cursor, hist_a, hist_b, running,
                  sem):
        cid = lax.axis_index("c")
        sid = lax.axis_index("s")
        wid = sid * NC + cid

        iota = lax.iota(jnp.int32, L)

        def row_body(t, acc):
            row = wid * rows_per_worker + t
            for pass_idx, (hist, hist_next) in enumerate(
                    ((hist_a, hist_b), (hist_b, hist_a), (hist_a, None))):
                _row_pass(row, pass_idx, pre_hbm, lab_hbm, key_ping,
                          pre_ping, key_out, pre_out, stage_k, stage_p,
                          cursor, hist, hist_next, running, sem)

            def epi_body(i, acc):
                for u in range(4):
                    base = (i * 4 + u) * L
                    a16 = pre_out[pl.ds(base, L)]
                    nxt = jnp.minimum(iota + (base + 1), jnp.int32(N - 1))
                    b16 = plsc.load_gather(pre_out, [nxt])
                    acc = acc + jnp.maximum(a16 - b16, 0.0)
                return acc

            return lax.fori_loop(0, N // L // 4, epi_body, acc)

        acc = lax.fori_loop(0, rows_per_worker, row_body,
                            jnp.zeros((L,), jnp.float32))

        stage_p[0, pl.ds(0, L)] = acc
        pltpu.sync_copy(stage_p.at[0, pl.ds(0, L)], out_hbm.at[wid])

    return sc_kernel


@jax.jit
def kernel(uncertainty_pre, uncertainty_label):
    b, n = uncertainty_pre.shape
    assert n == N and b % NWORK == 0
    out, _, _ = _make_sc_kernel(b)(uncertainty_pre, uncertainty_label)
    return jnp.sum(out) / b


# speedup vs baseline: 5.8786x; 1.0061x over previous
import functools

import jax
import jax.numpy as jnp
from jax import lax
from jax.experimental import pallas as pl
from jax.experimental.pallas import tpu as pltpu
from jax.experimental.pallas import tpu_sc as plsc

N = 32768
W = 4096
NWIN = N // W
GRP = 4
NGRP = NWIN // GRP
HSLOT = NWIN // 2
NC = 2
NS = 16
NWORK = NC * NS
L = 16

PASSES = ((0, 2048), (11, 2048), (22, 1024))


def _to_key(l16):
    bits = plsc.bitcast(l16, jnp.int32)
    return jnp.where(bits < 0, bits ^ jnp.int32(-1),
                     bits | jnp.int32(-2147483648))


def _digit(k16, shift, nbins):
    if shift == 0:
        return k16 & jnp.int32(nbins - 1)
    d = lax.shift_right_logical(k16, jnp.int32(shift))
    if shift + (nbins.bit_length() - 1) < 32:
        d = d & jnp.int32(nbins - 1)
    return d


def _drain(cps):
    for cp in cps:
        cp.wait()


def _row_pass(row, pass_idx, pre_hbm, lab_hbm, key_ping, pre_ping, key_out,
              pre_out, stage_k, stage_p, cursor, hist, hist_next, running,
              sem):
    shift, nbins = PASSES[pass_idx]
    nbv = nbins // L
    zeros16 = jnp.zeros((L,), jnp.int32)

    def zero_body(i, _):
        for h in range(HSLOT):
            if pass_idx == 0:
                hist[pl.ds(h * 2048 + i * L, L)] = zeros16
            if hist_next is not None:
                hist_next[pl.ds(h * 2048 + i * L, L)] = zeros16
        return 0

    lax.fori_loop(0, 2048 // L, zero_body, 0, unroll=2)

    def hist_group(g, _):
        cps = []
        for wl in range(GRP):
            w = g * GRP + wl
            cps.append(pltpu.async_copy(
                lab_hbm.at[row, pl.ds(w * W, W)], stage_p.at[wl], sem))
        _drain(cps)

        def hist_body(i, _):
            for wl in range(GRP):
                w = g * GRP + wl
                shamt = (w // HSLOT) * 16
                incr16 = zeros16 + lax.shift_left(jnp.int32(1), shamt)
                off = (w % HSLOT) * 2048
                l16 = stage_p[wl, pl.ds(i * L, L)]
                k16 = _to_key(l16)
                stage_k[wl, pl.ds(i * L, L)] = k16
                d = _digit(k16, shift, nbins)
                plsc.addupdate_scatter(hist, [d + off], incr16)
            return 0

        lax.fori_loop(0, W // L, hist_body, 0)

        cps = []
        for wl in range(GRP):
            w = g * GRP + wl
            cps.append(pltpu.async_copy(
                stage_k.at[wl], key_ping.at[row, pl.ds(w * W, W)], sem))
        _drain(cps)
        return 0

    if pass_idx == 0:
        lax.fori_loop(0, NGRP, hist_group, 0)

    def scan_body(i, carry):
        tot_pairs = hist[pl.ds(i * L, L)]
        for h in range(1, HSLOT):
            tot_pairs = tot_pairs + hist[pl.ds(h * 2048 + i * L, L)]
        tot16 = (tot_pairs & jnp.int32(0xFFFF)) + lax.shift_right_logical(
            tot_pairs, jnp.int32(16))
        incl = plsc.cumsum(tot16)
        running[pl.ds(i * L, L)] = incl - tot16 + carry
        return carry + jnp.sum(tot16)

    lax.fori_loop(0, nbv, scan_body, jnp.int32(0))

    def perm_group(g, _):
        cps = []
        for wl in range(GRP):
            w = g * GRP + wl
            cps.append(pltpu.async_copy(
                key_ping.at[row, pl.ds(w * W, W)], stage_k.at[wl], sem))
            if pass_idx == 0:
                cps.append(pltpu.async_copy(
                    pre_hbm.at[row, pl.ds(w * W, W)], stage_p.at[wl], sem))
            else:
                cps.append(pltpu.async_copy(
                    pre_ping.at[row, pl.ds(w * W, W)], stage_p.at[wl], sem))
        _drain(cps)

        def build_body(i, _):
            run16 = running[pl.ds(i * L, L)]
            for wl in range(GRP):
                w = g * GRP + wl
                shamt = (w // HSLOT) * 16
                cnt_pair = hist[pl.ds((w % HSLOT) * 2048 + i * L, L)]
                cnt16 = lax.shift_right_logical(cnt_pair, shamt) \
                    & jnp.int32(0xFFFF)
                cursor[pl.ds(wl * nbins + i * L, L)] = run16
                run16 = run16 + cnt16
            running[pl.ds(i * L, L)] = run16
            return 0

        lax.fori_loop(0, nbv, build_body, 0)

        ones_lo = zeros16 + jnp.int32(1)
        if hist_next is not None:
            nshift, nnbins = PASSES[pass_idx + 1]

        def perm_body(i, _):
            for wl in range(GRP):
                k16 = stage_k[wl, pl.ds(i * L, L)]
                d = _digit(k16, shift, nbins)
                cnt, last = plsc.scan_count(d)
                cidx = d + jnp.int32(wl * nbins)
                cur = plsc.load_gather(cursor, [cidx])
                pos = cur + (cnt - jnp.int32(1))
                if pass_idx != 2:
                    plsc.store_scatter(key_out, [pos], k16)
                p16 = stage_p[wl, pl.ds(i * L, L)]
                plsc.store_scatter(pre_out, [pos], p16)
                plsc.store_scatter(cursor, [cidx], cur + cnt, mask=last)
                if hist_next is not None:
                    dnext = _digit(k16, nshift, nnbins)
                    wdest = lax.shift_right_logical(pos, jnp.int32(12))
                    nidx = lax.shift_left(wdest & jnp.int32(HSLOT - 1),
                                          jnp.int32(11)) + dnext
                    shamt16 = lax.shift_left(
                        lax.shift_right_logical(wdest, jnp.int32(2)),
                        jnp.int32(4))
                    incr16 = lax.shift_left(ones_lo, shamt16)
                    plsc.addupdate_scatter(hist_next, [nidx], incr16)
            return 0

        lax.fori_loop(0, W // L, perm_body, 0)
        return 0

    lax.fori_loop(0, NGRP, perm_group, 0)

    if pass_idx != 2:
        cps = [pltpu.async_copy(key_out, key_ping.at[row], sem),
               pltpu.async_copy(pre_out, pre_ping.at[row], sem)]
        _drain(cps)


def _make_sc_kernel(batch):
    rows_per_worker = batch // NWORK
    mesh = plsc.VectorSubcoreMesh(core_axis_name="c", subcore_axis_name="s")

    @functools.partial(
        pl.kernel,
        mesh=mesh,
        out_type=(
            jax.ShapeDtypeStruct((NWORK, L), jnp.float32),
            jax.ShapeDtypeStruct((batch, N), jnp.int32),
            jax.ShapeDtypeStruct((batch, N), jnp.float32),
        ),
        scratch_types=[
            pltpu.VMEM((N,), jnp.int32),
            pltpu.VMEM((N,), jnp.float32),
            pltpu.VMEM((GRP, W), jnp.int32),
            pltpu.VMEM((GRP, W), jnp.float32),
            pltpu.VMEM((GRP * 2048,), jnp.int32),
            pltpu.VMEM((HSLOT * 2048,), jnp.int32),
            pltpu.VMEM((HSLOT * 2048,), jnp.int32),
            pltpu.VMEM((2048,), jnp.int32),
            pltpu.SemaphoreType.DMA,
        ],
        compiler_params=pltpu.CompilerParams(needs_layout_passes=False),
    )
    def sc_kernel(pre_hbm, lab_hbm, out_hbm, key_ping, pre_ping, key_out,
                  pre_out, stage_k, stage_p, cursor, hist_a, hist_b, running,
                  sem):
        cid = lax.axis_index("c")
        sid = lax.axis_index("s")
        wid = sid * NC + cid

        iota = lax.iota(jnp.int32, L)

        def row_body(t, acc):
            row = wid * rows_per_worker + t
            for pass_idx, (hist, hist_next) in enumerate(
                    ((hist_a, hist_b), (hist_b, hist_a), (hist_a, None))):
                _row_pass(row, pass_idx, pre_hbm, lab_hbm, key_ping,
                          pre_ping, key_out, pre_out, stage_k, stage_p,
                          cursor, hist, hist_next, running, sem)

            def epi_body(i, acc):
                for u in range(4):
                    base = (i * 4 + u) * L
                    a16 = pre_out[pl.ds(base, L)]
                    nxt = jnp.minimum(iota + (base + 1), jnp.int32(N - 1))
                    b16 = plsc.load_gather(pre_out, [nxt])
                    acc = acc + jnp.maximum(a16 - b16, 0.0)
                return acc

            return lax.fori_loop(0, N // L // 4, epi_body, acc)

        acc = lax.fori_loop(0, rows_per_worker, row_body,
                            jnp.zeros((L,), jnp.float32))

        stage_p[0, pl.ds(0, L)] = acc
        pltpu.sync_copy(stage_p.at[0, pl.ds(0, L)], out_hbm.at[wid])

    return sc_kernel


@jax.jit
def kernel(uncertainty_pre, uncertainty_label):
    b, n = uncertainty_pre.shape
    assert n == N and b % NWORK == 0
    out, _, _ = _make_sc_kernel(b)(uncertainty_pre, uncertainty_label)
    return jnp.sum(out) / b


# speedup vs baseline: 5.8862x; 1.0013x over previous
import functools

import jax
import jax.numpy as jnp
from jax import lax
from jax.experimental import pallas as pl
from jax.experimental.pallas import tpu as pltpu
from jax.experimental.pallas import tpu_sc as plsc

N = 32768
W = 4096
NWIN = N // W
GRP = 4
NGRP = NWIN // GRP
HSLOT = NWIN // 2
NC = 2
NS = 16
NWORK = NC * NS
L = 16

PASSES = ((0, 2048), (11, 2048), (22, 1024))


def _to_key(l16):
    bits = plsc.bitcast(l16, jnp.int32)
    return jnp.where(bits < 0, bits ^ jnp.int32(-1),
                     bits | jnp.int32(-2147483648))


def _digit(k16, shift, nbins):
    if shift == 0:
        return k16 & jnp.int32(nbins - 1)
    d = lax.shift_right_logical(k16, jnp.int32(shift))
    if shift + (nbins.bit_length() - 1) < 32:
        d = d & jnp.int32(nbins - 1)
    return d


def _drain(cps):
    for cp in cps:
        cp.wait()


def _row_pass(row, pass_idx, pre_hbm, lab_hbm, key_ping, pre_ping, key_out,
              pre_out, stage_k, stage_p, cursor, hist, hist_next, running,
              sem):
    shift, nbins = PASSES[pass_idx]
    nbv = nbins // L
    zeros16 = jnp.zeros((L,), jnp.int32)

    def zero_body(i, _):
        for h in range(HSLOT):
            if pass_idx == 0:
                hist[pl.ds(h * 2048 + i * L, L)] = zeros16
            if hist_next is not None:
                hist_next[pl.ds(h * 2048 + i * L, L)] = zeros16
        return 0

    lax.fori_loop(0, 2048 // L, zero_body, 0, unroll=2)

    def hist_group(g, _):
        cps = []
        for wl in range(GRP):
            w = g * GRP + wl
            cps.append(pltpu.async_copy(
                lab_hbm.at[row, pl.ds(w * W, W)], stage_p.at[wl], sem))
        _drain(cps)

        def hist_body(i, _):
            for wl in range(GRP):
                w = g * GRP + wl
                shamt = (w // HSLOT) * 16
                incr16 = zeros16 + lax.shift_left(jnp.int32(1), shamt)
                off = (w % HSLOT) * 2048
                l16 = stage_p[wl, pl.ds(i * L, L)]
                k16 = _to_key(l16)
                stage_k[wl, pl.ds(i * L, L)] = k16
                d = _digit(k16, shift, nbins)
                plsc.addupdate_scatter(hist, [d + off], incr16)
            return 0

        lax.fori_loop(0, W // L, hist_body, 0, unroll=2)

        cps = []
        for wl in range(GRP):
            w = g * GRP + wl
            cps.append(pltpu.async_copy(
                stage_k.at[wl], key_ping.at[row, pl.ds(w * W, W)], sem))
        _drain(cps)
        return 0

    if pass_idx == 0:
        lax.fori_loop(0, NGRP, hist_group, 0)

    def scan_body(i, carry):
        tot_pairs = hist[pl.ds(i * L, L)]
        for h in range(1, HSLOT):
            tot_pairs = tot_pairs + hist[pl.ds(h * 2048 + i * L, L)]
        tot16 = (tot_pairs & jnp.int32(0xFFFF)) + lax.shift_right_logical(
            tot_pairs, jnp.int32(16))
        incl = plsc.cumsum(tot16)
        running[pl.ds(i * L, L)] = incl - tot16 + carry
        return carry + jnp.sum(tot16)

    lax.fori_loop(0, nbv, scan_body, jnp.int32(0), unroll=2)

    def perm_group(g, _):
        cps = []
        for wl in range(GRP):
            w = g * GRP + wl
            cps.append(pltpu.async_copy(
                key_ping.at[row, pl.ds(w * W, W)], stage_k.at[wl], sem))
            if pass_idx == 0:
                cps.append(pltpu.async_copy(
                    pre_hbm.at[row, pl.ds(w * W, W)], stage_p.at[wl], sem))
            else:
                cps.append(pltpu.async_copy(
                    pre_ping.at[row, pl.ds(w * W, W)], stage_p.at[wl], sem))
        _drain(cps)

        def build_body(i, _):
            run16 = running[pl.ds(i * L, L)]
            for wl in range(GRP):
                w = g * GRP + wl
                shamt = (w // HSLOT) * 16
                cnt_pair = hist[pl.ds((w % HSLOT) * 2048 + i * L, L)]
                cnt16 = lax.shift_right_logical(cnt_pair, shamt) \
                    & jnp.int32(0xFFFF)
                cursor[pl.ds(wl * nbins + i * L, L)] = run16
                run16 = run16 + cnt16
            running[pl.ds(i * L, L)] = run16
            return 0

        lax.fori_loop(0, nbv, build_body, 0, unroll=2)

        ones_lo = zeros16 + jnp.int32(1)
        if hist_next is not None:
            nshift, nnbins = PASSES[pass_idx + 1]

        def perm_body(i, _):
            for wl in range(GRP):
                k16 = stage_k[wl, pl.ds(i * L, L)]
                d = _digit(k16, shift, nbins)
                cnt, last = plsc.scan_count(d)
                cidx = d + jnp.int32(wl * nbins)
                cur = plsc.load_gather(cursor, [cidx])
                pos = cur + (cnt - jnp.int32(1))
                if pass_idx != 2:
                    plsc.store_scatter(key_out, [pos], k16)
                p16 = stage_p[wl, pl.ds(i * L, L)]
                plsc.store_scatter(pre_out, [pos], p16)
                plsc.store_scatter(cursor, [cidx], cur + cnt, mask=last)
                if hist_next is not None:
                    dnext = _digit(k16, nshift, nnbins)
                    wdest = lax.shift_right_logical(pos, jnp.int32(12))
                    nidx = lax.shift_left(wdest & jnp.int32(HSLOT - 1),
                                          jnp.int32(11)) + dnext
                    shamt16 = lax.shift_left(
                        lax.shift_right_logical(wdest, jnp.int32(2)),
                        jnp.int32(4))
                    incr16 = lax.shift_left(ones_lo, shamt16)
                    plsc.addupdate_scatter(hist_next, [nidx], incr16)
            return 0

        lax.fori_loop(0, W // L, perm_body, 0, unroll=2)
        return 0

    lax.fori_loop(0, NGRP, perm_group, 0)

    if pass_idx != 2:
        cps = [pltpu.async_copy(key_out, key_ping.at[row], sem),
               pltpu.async_copy(pre_out, pre_ping.at[row], sem)]
        _drain(cps)


def _make_sc_kernel(batch):
    rows_per_worker = batch // NWORK
    mesh = plsc.VectorSubcoreMesh(core_axis_name="c", subcore_axis_name="s")

    @functools.partial(
        pl.kernel,
        mesh=mesh,
        out_type=(
            jax.ShapeDtypeStruct((NWORK, L), jnp.float32),
            jax.ShapeDtypeStruct((batch, N), jnp.int32),
            jax.ShapeDtypeStruct((batch, N), jnp.float32),
        ),
        scratch_types=[
            pltpu.VMEM((N,), jnp.int32),
            pltpu.VMEM((N,), jnp.float32),
            pltpu.VMEM((GRP, W), jnp.int32),
            pltpu.VMEM((GRP, W), jnp.float32),
            pltpu.VMEM((GRP * 2048,), jnp.int32),
            pltpu.VMEM((HSLOT * 2048,), jnp.int32),
            pltpu.VMEM((HSLOT * 2048,), jnp.int32),
            pltpu.VMEM((2048,), jnp.int32),
            pltpu.SemaphoreType.DMA,
        ],
        compiler_params=pltpu.CompilerParams(needs_layout_passes=False),
    )
    def sc_kernel(pre_hbm, lab_hbm, out_hbm, key_ping, pre_ping, key_out,
                  pre_out, stage_k, stage_p, cursor, hist_a, hist_b, running,
                  sem):
        cid = lax.axis_index("c")
        sid = lax.axis_index("s")
        wid = sid * NC + cid

        iota = lax.iota(jnp.int32, L)

        def row_body(t, acc):
            row = wid * rows_per_worker + t
            for pass_idx, (hist, hist_next) in enumerate(
                    ((hist_a, hist_b), (hist_b, hist_a), (hist_a, None))):
                _row_pass(row, pass_idx, pre_hbm, lab_hbm, key_ping,
                          pre_ping, key_out, pre_out, stage_k, stage_p,
                          cursor, hist, hist_next, running, sem)

            def epi_body(i, acc):
                for u in range(4):
                    base = (i * 4 + u) * L
                    a16 = pre_out[pl.ds(base, L)]
                    nxt = jnp.minimum(iota + (base + 1), jnp.int32(N - 1))
                    b16 = plsc.load_gather(pre_out, [nxt])
                    acc = acc + jnp.maximum(a16 - b16, 0.0)
                return acc

            return lax.fori_loop(0, N // L // 4, epi_body, acc)

        acc = lax.fori_loop(0, rows_per_worker, row_body,
                            jnp.zeros((L,), jnp.float32))

        stage_p[0, pl.ds(0, L)] = acc
        pltpu.sync_copy(stage_p.at[0, pl.ds(0, L)], out_hbm.at[wid])

    return sc_kernel


@jax.jit
def kernel(uncertainty_pre, uncertainty_label):
    b, n = uncertainty_pre.shape
    assert n == N and b % NWORK == 0
    out, _, _ = _make_sc_kernel(b)(uncertainty_pre, uncertainty_label)
    return jnp.sum(out) / b
